# speedup vs baseline: 3.9748x; 3.9748x over previous
import jax
import jax.numpy as jnp
from jax.experimental import pallas as pl
from jax.experimental.pallas import tpu as pltpu

HIDDEN = 1024
ROWS = 4 * 8192
BLOCK = 2048


def _body(seg_ref, tab_ref, out_ref):
    seg = seg_ref[...]
    t0 = tab_ref[0:1, :]
    t1 = tab_ref[1:2, :]
    out_ref[...] = jnp.where(seg == 0, t0, t1)


def kernel(segments, table):
    seg = segments.reshape(ROWS, 1).astype(jnp.int32)
    out = pl.pallas_call(
        _body,
        grid=(ROWS // BLOCK,),
        in_specs=[
            pl.BlockSpec((BLOCK, 1), lambda i: (i, 0)),
            pl.BlockSpec((2, HIDDEN), lambda i: (0, 0)),
        ],
        out_specs=pl.BlockSpec((BLOCK, HIDDEN), lambda i: (i, 0)),
        out_shape=jax.ShapeDtypeStruct((ROWS, HIDDEN), jnp.float32),
    )(seg, table)
    return out.reshape(segments.shape[0], segments.shape[1], HIDDEN)
